# baseline (speedup 1.0000x reference)
import functools

import jax
import jax.numpy as jnp
from jax import lax
from jax.experimental import pallas as pl
from jax.experimental.pallas import tpu as pltpu
from jax.experimental.pallas import tpu_sc as plsc

VIEWS = 3
B = 2
C = 128
H = 64
W = 64
N = 50000
NP = 50176
SLOTS = VIEWS * B * H * W
NWORK = 32
CHUNK = NP // NWORK
SPT = SLOTS // NWORK
PAD_SLOT = 32000
GPAD = SLOTS
DENSE_PAD = NP
GTOT = 32 * 37 * 128
BM = 1792
NB = NP // BM

def _mesh():
    return plsc.VectorSubcoreMesh(core_axis_name="c", subcore_axis_name="s")

_GDN = lax.GatherDimensionNumbers(
    offset_dims=(), collapsed_slice_dims=(0,), start_index_map=(0,))


def _take16(x, idx):
    return lax.gather(x, idx.reshape(16, 1), _GDN, slice_sizes=(1,),
                      mode=lax.GatherScatterMode.PROMISE_IN_BOUNDS)


def _winner_phase1(slots, init):
    @functools.partial(
        pl.kernel, mesh=_mesh(),
        compiler_params=pltpu.CompilerParams(needs_layout_passes=False),
        out_type=jax.ShapeDtypeStruct((NWORK * SLOTS,), jnp.int32),
        scratch_types=[
            pltpu.VMEM((CHUNK,), jnp.int32),
            pltpu.VMEM((SLOTS,), jnp.int32),
        ],
    )
    def k(slots_hbm, init_hbm, out_hbm, chunk_v, win_v):
        wid = lax.axis_index("s") * 2 + lax.axis_index("c")
        pltpu.sync_copy(init_hbm, win_v)
        lanes = lax.iota(jnp.int32, 16)
        for v in range(VIEWS):
            pltpu.sync_copy(
                slots_hbm.at[pl.ds(v * NP + wid * CHUNK, CHUNK)], chunk_v)
            base_n = wid * CHUNK

            def body(i, _):
                s = chunk_v[pl.ds(i * 16, 16)]
                n = base_n + i * 16 + lanes
                dup = lanes < 0
                for sh in range(1, 16):
                    hi = _take16(s, jnp.minimum(lanes + sh, 15))
                    dup = dup | ((hi == s) & (lanes + sh <= 15))
                ok = (~dup) & (s < SLOTS)
                plsc.store_scatter(win_v, [s], n, mask=ok)
                return 0

            lax.fori_loop(0, CHUNK // 16, body, 0)
        pltpu.sync_copy(win_v, out_hbm.at[pl.ds(wid * SLOTS, SLOTS)])

    return k(slots, init)


def _winner_reduce_gather(w32, a_ext):
    @functools.partial(
        pl.kernel, mesh=_mesh(),
        compiler_params=pltpu.CompilerParams(needs_layout_passes=False),
        out_type=jax.ShapeDtypeStruct((SLOTS, C), jnp.float32),
        scratch_types=[
            pltpu.VMEM((NWORK, SPT), jnp.int32),
            pltpu.VMEM((6, 128), jnp.int32),
            pltpu.VMEM((SPT, C), jnp.float32),
            pltpu.SemaphoreType.DMA,
        ],
    )
    def k(w_hbm, a_hbm, out_hbm, red_v, idx_v, rows_v, sem):
        wid = lax.axis_index("s") * 2 + lax.axis_index("c")
        base = wid * SPT
        pltpu.sync_copy(w_hbm.at[:, pl.ds(base, SPT)], red_v)
        for r in range(6):
            def body(c2, _):
                off = (r * 8 + c2) * 16
                m = red_v[0, pl.ds(off, 16)]
                for j in range(1, NWORK):
                    m = jnp.maximum(m, red_v[j, pl.ds(off, 16)])
                m = jnp.where(m < 0, DENSE_PAD, m)
                idx_v[r, pl.ds(c2 * 16, 16)] = m
                return 0

            lax.fori_loop(0, 8, body, 0)
        for r in range(6):
            pltpu.async_copy(a_hbm.at[idx_v.at[r]],
                             rows_v.at[pl.ds(r * 128, 128)], sem).wait()
        pltpu.sync_copy(rows_v, out_hbm.at[pl.ds(base, SPT)])

    return k(w32, a_ext)


def _sc_gather(tab, idx):
    @functools.partial(
        pl.kernel, mesh=_mesh(),
        compiler_params=pltpu.CompilerParams(needs_layout_passes=False),
        out_type=jax.ShapeDtypeStruct((GTOT, C), jnp.float32),
        scratch_types=[
            pltpu.VMEM((48, 128), jnp.int32),
            pltpu.VMEM((128, C), jnp.float32),
            pltpu.VMEM((128, C), jnp.float32),
            pltpu.SemaphoreType.DMA,
            pltpu.SemaphoreType.DMA,
        ],
    )
    def k(tab_hbm, idx_hbm, out_hbm, idx_v, buf0, buf1, sem0, sem1):
        wid = lax.axis_index("s") * 2 + lax.axis_index("c")
        pltpu.sync_copy(idx_hbm.at[wid], idx_v)
        bufs = (buf0, buf1)
        sems = (sem0, sem1)
        cps = [None, None]
        for j in range(37):
            p = j % 2
            cps[p] = pltpu.async_copy(tab_hbm.at[idx_v.at[j]], bufs[p],
                                      sems[p])
            if j > 0:
                q = (j - 1) % 2
                cps[q].wait()
                pltpu.sync_copy(
                    bufs[q],
                    out_hbm.at[pl.ds((wid * 37 + j - 1) * 128, 128)])
        cps[36 % 2].wait()
        pltpu.sync_copy(bufs[36 % 2],
                        out_hbm.at[pl.ds((wid * 37 + 36) * 128, 128)])

    return k(tab, idx)


def _mm_stats(terms, n_valid):
    nt = len(terms)

    def body(*refs):
        i = pl.program_id(0)
        xr = refs[0:nt]
        wr = refs[nt:2 * nt]
        pos = 2 * nt
        affr = []
        for t in range(nt):
            if terms[t][3] is not None:
                affr.append((refs[pos], refs[pos + 1]))
                pos += 2
            else:
                affr.append(None)
        y_ref, st_ref, acc_ref = refs[pos], refs[pos + 1], refs[pos + 2]
        y = jnp.zeros((BM, C), jnp.float32)
        for t in range(nt):
            xv = xr[t][...]
            if affr[t] is not None:
                s, b = affr[t]
                xv = jnp.maximum(xv * s[...] + b[...], 0.0)
            y = y + jnp.dot(xv, wr[t][...],
                            preferred_element_type=jnp.float32)
        y_ref[...] = y
        rows = i * BM + lax.broadcasted_iota(jnp.int32, (BM, 1), 0)
        ym = jnp.where(rows < n_valid, y, 0.0)

        @pl.when(i == 0)
        def _():
            acc_ref[...] = jnp.zeros_like(acc_ref)

        acc_ref[0, :] += jnp.sum(ym, axis=0)
        acc_ref[1, :] += jnp.sum(ym * ym, axis=0)

        @pl.when(i == NB - 1)
        def _():
            st_ref[...] = acc_ref[...]

    in_specs = []
    args = []
    for (x, off, w, aff) in terms:
        in_specs.append(pl.BlockSpec((BM, C), lambda i, o=off: (i + o, 0)))
        args.append(x)
    for (x, off, w, aff) in terms:
        in_specs.append(pl.BlockSpec((C, C), lambda i: (0, 0)))
        args.append(w)
    for (x, off, w, aff) in terms:
        if aff is not None:
            in_specs.append(pl.BlockSpec((1, C), lambda i: (0, 0)))
            in_specs.append(pl.BlockSpec((1, C), lambda i: (0, 0)))
            args.append(aff[0])
            args.append(aff[1])

    return pl.pallas_call(
        body,
        grid=(NB,),
        in_specs=in_specs,
        out_specs=[pl.BlockSpec((BM, C), lambda i: (i, 0)),
                   pl.BlockSpec((2, C), lambda i: (0, 0))],
        out_shape=[jax.ShapeDtypeStruct((NP, C), jnp.float32),
                   jax.ShapeDtypeStruct((2, C), jnp.float32)],
        scratch_shapes=[pltpu.VMEM((2, C), jnp.float32)],
    )(*args)


def _ew_affine_relu(x, s, b, bm):
    rows = x.shape[0]
    nb = rows // bm

    def body(x_ref, s_ref, b_ref, o_ref):
        o_ref[...] = jnp.maximum(x_ref[...] * s_ref[...] + b_ref[...], 0.0)

    return pl.pallas_call(
        body,
        grid=(nb,),
        in_specs=[pl.BlockSpec((bm, C), lambda i: (i, 0)),
                  pl.BlockSpec((1, C), lambda i: (0, 0)),
                  pl.BlockSpec((1, C), lambda i: (0, 0))],
        out_specs=pl.BlockSpec((bm, C), lambda i: (i, 0)),
        out_shape=jax.ShapeDtypeStruct((rows, C), jnp.float32),
    )(x, s, b)


FL = 4224
XL = 4360


def _conv_stats(xflat, wr):
    def body(x_ref, w_ref, y_ref, st_ref, acc_ref):
        i = pl.program_id(0)
        y = jnp.zeros((FL, C), jnp.float32)
        for dy in range(3):
            for dx in range(3):
                off = 66 * dy + dx
                t = dy * 3 + dx
                xs = x_ref[0, pl.ds(off, FL), :]
                y = y + jnp.dot(xs, w_ref[pl.ds(t * 256, 256), :],
                                preferred_element_type=jnp.float32)
        y_ref[0] = y
        col = lax.broadcasted_iota(jnp.int32, (FL, C), 0) % 66
        ym = jnp.where(col < 64, y, 0.0)

        @pl.when(i == 0)
        def _():
            acc_ref[...] = jnp.zeros_like(acc_ref)

        acc_ref[0, :] += jnp.sum(ym, axis=0)
        acc_ref[1, :] += jnp.sum(ym * ym, axis=0)

        @pl.when(i == 5)
        def _():
            st_ref[...] = acc_ref[...]

    return pl.pallas_call(
        body,
        grid=(6,),
        in_specs=[pl.BlockSpec((1, XL, 256), lambda i: (i, 0, 0)),
                  pl.BlockSpec((2304, C), lambda i: (0, 0))],
        out_specs=[pl.BlockSpec((1, FL, C), lambda i: (i, 0, 0)),
                   pl.BlockSpec((2, C), lambda i: (0, 0))],
        out_shape=[jax.ShapeDtypeStruct((6, FL, C), jnp.float32),
                   jax.ShapeDtypeStruct((2, C), jnp.float32)],
        scratch_shapes=[pltpu.VMEM((2, C), jnp.float32)],
    )(xflat, wr)


def _finalize(st, cnt, g, b):
    mean = st[0] / cnt
    var = st[1] / cnt - mean * mean
    scale = g / jnp.sqrt(var + 1e-5)
    bias = b - mean * scale
    return scale.reshape(1, C), bias.reshape(1, C)


def kernel(feat_2d_all, feat_3d_F, links, W_sep, g_sep, b_sep,
           W_vf1, g_vf1, b_vf1, W_vf2, g_vf2, b_vf2,
           W_f3d, g_f3d, b_f3d, W_conv, g_2d, b_2d):
    f32 = jnp.float32
    f2d_nhwc = feat_2d_all.reshape(VIEWS * B, C, H, W).transpose(0, 2, 3, 1)
    f2v_ext = jnp.concatenate(
        [f2d_nhwc.reshape(SLOTS, C), jnp.zeros((8, C), f32)], axis=0)
    f3d_pad = jnp.concatenate(
        [feat_3d_F, jnp.zeros((NP - N, C), f32)], axis=0)

    varr = jnp.arange(VIEWS, dtype=jnp.int32)[None, :]
    bi = links[:, 0, :]
    hi = links[:, 1, :]
    wi = links[:, 2, :]
    fl = links[:, 3, :]
    slot = ((varr * B + bi) * H + hi) * W + wi
    gidx = jnp.where(fl == 1, slot, GPAD)

    slot_t = jnp.full((VIEWS, NP), PAD_SLOT, jnp.int32)
    slot_t = slot_t.at[:, :N].set(slot.T).reshape(-1)
    gidx_flat = jnp.full((GTOT,), GPAD, jnp.int32)
    gidx_flat = gidx_flat.at[:VIEWS * NP].set(
        jnp.full((VIEWS, NP), GPAD, jnp.int32).at[:, :N].set(gidx.T).reshape(-1))
    gidx_rs = jnp.pad(gidx_flat.reshape(NWORK, 37, 128),
                      ((0, 0), (0, 11), (0, 0)), constant_values=GPAD)

    y_sep, st_sep = _mm_stats([(f3d_pad, 0, W_sep, None)], N)
    s_sep, o_sep = _finalize(st_sep, N, g_sep, b_sep)
    a_mat = _ew_affine_relu(y_sep, s_sep, o_sep, BM)
    a_ext = jnp.concatenate([a_mat, jnp.zeros((8, C), f32)], axis=0)

    init = jnp.full((SLOTS,), -1, jnp.int32)
    w32 = _winner_phase1(slot_t, init).reshape(NWORK, SLOTS)
    dense2d = _winner_reduce_gather(w32, a_ext)
    gact = _sc_gather(f2v_ext, gidx_rs)

    wv = W_vf1.reshape(VIEWS, C, C)
    y_vf1, st1 = _mm_stats(
        [(gact, 0, wv[0], None), (gact, NB, wv[1], None),
         (gact, 2 * NB, wv[2], None)], N)
    s1, o1 = _finalize(st1, N, g_vf1, b_vf1)
    y_vf2, st2 = _mm_stats([(y_vf1, 0, W_vf2, (s1, o1))], N)
    s2, o2 = _finalize(st2, N, g_vf2, b_vf2)
    wf = W_f3d.reshape(2, C, C)
    y_f3d, st3 = _mm_stats(
        [(f3d_pad, 0, wf[0], None), (y_vf2, 0, wf[1], (s2, o2))], N)
    s3, o3 = _finalize(st3, N, g_f3d, b_f3d)
    fused_3d = _ew_affine_relu(y_f3d, s3, o3, BM)[:N]

    cat = jnp.concatenate(
        [f2d_nhwc, dense2d.reshape(VIEWS * B, H, W, C)], axis=-1)
    xpad = jnp.pad(cat, ((0, 0), (1, 1), (1, 1), (0, 0)))
    xflat = jnp.pad(xpad.reshape(VIEWS * B, 66 * 66, 2 * C),
                    ((0, 0), (0, XL - 66 * 66), (0, 0)))
    wr = W_conv.transpose(2, 3, 1, 0).reshape(9 * 2 * C, C)
    y_cw, stc = _conv_stats(xflat, wr)
    sc_, oc_ = _finalize(stc, 6 * H * W, g_2d, b_2d)
    y2 = _ew_affine_relu(y_cw.reshape(6 * FL, C), sc_, oc_, FL)
    fused_2d = (y2.reshape(VIEWS * B, H, 66, C)[:, :, :64, :]
                .transpose(0, 3, 1, 2))
    return fused_3d, fused_2d


# speedup vs baseline: 1.0005x; 1.0005x over previous
import functools

import jax
import jax.numpy as jnp
from jax import lax
from jax.experimental import pallas as pl
from jax.experimental.pallas import tpu as pltpu
from jax.experimental.pallas import tpu_sc as plsc

VIEWS = 3
B = 2
C = 128
H = 64
W = 64
N = 50000
NP = 50176
SLOTS = VIEWS * B * H * W
NWORK = 32
CHUNK = NP // NWORK
SPT = SLOTS // NWORK
PAD_SLOT = 32000
GPAD = SLOTS
DENSE_PAD = NP
GTOT = 32 * 37 * 128
BM = 1792
NB = NP // BM

def _mesh():
    return plsc.VectorSubcoreMesh(core_axis_name="c", subcore_axis_name="s")


def _winner_phase1(slots, init):
    @functools.partial(
        pl.kernel, mesh=_mesh(),
        compiler_params=pltpu.CompilerParams(needs_layout_passes=False),
        out_type=jax.ShapeDtypeStruct((NWORK * SLOTS,), jnp.int32),
        scratch_types=[
            pltpu.VMEM((CHUNK,), jnp.int32),
            pltpu.VMEM((SLOTS,), jnp.int32),
        ],
    )
    def k(slots_hbm, init_hbm, out_hbm, chunk_v, win_v):
        wid = lax.axis_index("s") * 2 + lax.axis_index("c")
        pltpu.sync_copy(init_hbm, win_v)
        lanes = lax.iota(jnp.int32, 16)
        for v in range(VIEWS):
            pltpu.sync_copy(
                slots_hbm.at[pl.ds(v * NP + wid * CHUNK, CHUNK)], chunk_v)
            base_n = wid * CHUNK

            def body(i, _):
                s = chunk_v[pl.ds(i * 16, 16)]
                n = base_n + i * 16 + lanes
                valid = s < SLOTS
                _, last = plsc.scan_count(s, valid)
                plsc.store_scatter(win_v, [s], n, mask=last & valid)
                return 0

            lax.fori_loop(0, CHUNK // 16, body, 0)
        pltpu.sync_copy(win_v, out_hbm.at[pl.ds(wid * SLOTS, SLOTS)])

    return k(slots, init)


def _winner_reduce_gather(w32, a_ext):
    @functools.partial(
        pl.kernel, mesh=_mesh(),
        compiler_params=pltpu.CompilerParams(needs_layout_passes=False),
        out_type=jax.ShapeDtypeStruct((SLOTS, C), jnp.float32),
        scratch_types=[
            pltpu.VMEM((NWORK, SPT), jnp.int32),
            pltpu.VMEM((6, 128), jnp.int32),
            pltpu.VMEM((SPT, C), jnp.float32),
            pltpu.SemaphoreType.DMA,
        ],
    )
    def k(w_hbm, a_hbm, out_hbm, red_v, idx_v, rows_v, sem):
        wid = lax.axis_index("s") * 2 + lax.axis_index("c")
        base = wid * SPT
        pltpu.sync_copy(w_hbm.at[:, pl.ds(base, SPT)], red_v)
        for r in range(6):
            def body(c2, _):
                off = (r * 8 + c2) * 16
                m = red_v[0, pl.ds(off, 16)]
                for j in range(1, NWORK):
                    m = jnp.maximum(m, red_v[j, pl.ds(off, 16)])
                m = jnp.where(m < 0, DENSE_PAD, m)
                idx_v[r, pl.ds(c2 * 16, 16)] = m
                return 0

            lax.fori_loop(0, 8, body, 0)
        for r in range(6):
            pltpu.async_copy(a_hbm.at[idx_v.at[r]],
                             rows_v.at[pl.ds(r * 128, 128)], sem).wait()
        pltpu.sync_copy(rows_v, out_hbm.at[pl.ds(base, SPT)])

    return k(w32, a_ext)


def _sc_gather(tab, idx):
    @functools.partial(
        pl.kernel, mesh=_mesh(),
        compiler_params=pltpu.CompilerParams(needs_layout_passes=False),
        out_type=jax.ShapeDtypeStruct((GTOT, C), jnp.float32),
        scratch_types=[
            pltpu.VMEM((48, 128), jnp.int32),
            pltpu.VMEM((128, C), jnp.float32),
            pltpu.VMEM((128, C), jnp.float32),
            pltpu.SemaphoreType.DMA,
            pltpu.SemaphoreType.DMA,
        ],
    )
    def k(tab_hbm, idx_hbm, out_hbm, idx_v, buf0, buf1, sem0, sem1):
        wid = lax.axis_index("s") * 2 + lax.axis_index("c")
        pltpu.sync_copy(idx_hbm.at[wid], idx_v)
        bufs = (buf0, buf1)
        sems = (sem0, sem1)
        cps = [None, None]
        for j in range(37):
            p = j % 2
            cps[p] = pltpu.async_copy(tab_hbm.at[idx_v.at[j]], bufs[p],
                                      sems[p])
            if j > 0:
                q = (j - 1) % 2
                cps[q].wait()
                pltpu.sync_copy(
                    bufs[q],
                    out_hbm.at[pl.ds((wid * 37 + j - 1) * 128, 128)])
        cps[36 % 2].wait()
        pltpu.sync_copy(bufs[36 % 2],
                        out_hbm.at[pl.ds((wid * 37 + 36) * 128, 128)])

    return k(tab, idx)


def _mm_stats(terms, n_valid):
    nt = len(terms)

    def body(*refs):
        i = pl.program_id(0)
        xr = refs[0:nt]
        wr = refs[nt:2 * nt]
        pos = 2 * nt
        affr = []
        for t in range(nt):
            if terms[t][3] is not None:
                affr.append((refs[pos], refs[pos + 1]))
                pos += 2
            else:
                affr.append(None)
        y_ref, st_ref, acc_ref = refs[pos], refs[pos + 1], refs[pos + 2]
        y = jnp.zeros((BM, C), jnp.float32)
        for t in range(nt):
            xv = xr[t][...]
            if affr[t] is not None:
                s, b = affr[t]
                xv = jnp.maximum(xv * s[...] + b[...], 0.0)
            y = y + jnp.dot(xv, wr[t][...],
                            preferred_element_type=jnp.float32)
        y_ref[...] = y
        rows = i * BM + lax.broadcasted_iota(jnp.int32, (BM, 1), 0)
        ym = jnp.where(rows < n_valid, y, 0.0)

        @pl.when(i == 0)
        def _():
            acc_ref[...] = jnp.zeros_like(acc_ref)

        acc_ref[0, :] += jnp.sum(ym, axis=0)
        acc_ref[1, :] += jnp.sum(ym * ym, axis=0)

        @pl.when(i == NB - 1)
        def _():
            st_ref[...] = acc_ref[...]

    in_specs = []
    args = []
    for (x, off, w, aff) in terms:
        in_specs.append(pl.BlockSpec((BM, C), lambda i, o=off: (i + o, 0)))
        args.append(x)
    for (x, off, w, aff) in terms:
        in_specs.append(pl.BlockSpec((C, C), lambda i: (0, 0)))
        args.append(w)
    for (x, off, w, aff) in terms:
        if aff is not None:
            in_specs.append(pl.BlockSpec((1, C), lambda i: (0, 0)))
            in_specs.append(pl.BlockSpec((1, C), lambda i: (0, 0)))
            args.append(aff[0])
            args.append(aff[1])

    return pl.pallas_call(
        body,
        grid=(NB,),
        in_specs=in_specs,
        out_specs=[pl.BlockSpec((BM, C), lambda i: (i, 0)),
                   pl.BlockSpec((2, C), lambda i: (0, 0))],
        out_shape=[jax.ShapeDtypeStruct((NP, C), jnp.float32),
                   jax.ShapeDtypeStruct((2, C), jnp.float32)],
        scratch_shapes=[pltpu.VMEM((2, C), jnp.float32)],
    )(*args)


def _ew_affine_relu(x, s, b, bm):
    rows = x.shape[0]
    nb = rows // bm

    def body(x_ref, s_ref, b_ref, o_ref):
        o_ref[...] = jnp.maximum(x_ref[...] * s_ref[...] + b_ref[...], 0.0)

    return pl.pallas_call(
        body,
        grid=(nb,),
        in_specs=[pl.BlockSpec((bm, C), lambda i: (i, 0)),
                  pl.BlockSpec((1, C), lambda i: (0, 0)),
                  pl.BlockSpec((1, C), lambda i: (0, 0))],
        out_specs=pl.BlockSpec((bm, C), lambda i: (i, 0)),
        out_shape=jax.ShapeDtypeStruct((rows, C), jnp.float32),
    )(x, s, b)


FL = 4224
XL = 4360


def _conv_stats(xflat, wr):
    def body(x_ref, w_ref, y_ref, st_ref, acc_ref):
        i = pl.program_id(0)
        y = jnp.zeros((FL, C), jnp.float32)
        for dy in range(3):
            for dx in range(3):
                off = 66 * dy + dx
                t = dy * 3 + dx
                xs = x_ref[0, pl.ds(off, FL), :]
                y = y + jnp.dot(xs, w_ref[pl.ds(t * 256, 256), :],
                                preferred_element_type=jnp.float32)
        y_ref[0] = y
        col = lax.broadcasted_iota(jnp.int32, (FL, C), 0) % 66
        ym = jnp.where(col < 64, y, 0.0)

        @pl.when(i == 0)
        def _():
            acc_ref[...] = jnp.zeros_like(acc_ref)

        acc_ref[0, :] += jnp.sum(ym, axis=0)
        acc_ref[1, :] += jnp.sum(ym * ym, axis=0)

        @pl.when(i == 5)
        def _():
            st_ref[...] = acc_ref[...]

    return pl.pallas_call(
        body,
        grid=(6,),
        in_specs=[pl.BlockSpec((1, XL, 256), lambda i: (i, 0, 0)),
                  pl.BlockSpec((2304, C), lambda i: (0, 0))],
        out_specs=[pl.BlockSpec((1, FL, C), lambda i: (i, 0, 0)),
                   pl.BlockSpec((2, C), lambda i: (0, 0))],
        out_shape=[jax.ShapeDtypeStruct((6, FL, C), jnp.float32),
                   jax.ShapeDtypeStruct((2, C), jnp.float32)],
        scratch_shapes=[pltpu.VMEM((2, C), jnp.float32)],
    )(xflat, wr)


def _finalize(st, cnt, g, b):
    mean = st[0] / cnt
    var = st[1] / cnt - mean * mean
    scale = g / jnp.sqrt(var + 1e-5)
    bias = b - mean * scale
    return scale.reshape(1, C), bias.reshape(1, C)


def kernel(feat_2d_all, feat_3d_F, links, W_sep, g_sep, b_sep,
           W_vf1, g_vf1, b_vf1, W_vf2, g_vf2, b_vf2,
           W_f3d, g_f3d, b_f3d, W_conv, g_2d, b_2d):
    f32 = jnp.float32
    f2d_nhwc = feat_2d_all.reshape(VIEWS * B, C, H, W).transpose(0, 2, 3, 1)
    f2v_ext = jnp.concatenate(
        [f2d_nhwc.reshape(SLOTS, C), jnp.zeros((8, C), f32)], axis=0)
    f3d_pad = jnp.concatenate(
        [feat_3d_F, jnp.zeros((NP - N, C), f32)], axis=0)

    varr = jnp.arange(VIEWS, dtype=jnp.int32)[None, :]
    bi = links[:, 0, :]
    hi = links[:, 1, :]
    wi = links[:, 2, :]
    fl = links[:, 3, :]
    slot = ((varr * B + bi) * H + hi) * W + wi
    gidx = jnp.where(fl == 1, slot, GPAD)

    slot_t = jnp.full((VIEWS, NP), PAD_SLOT, jnp.int32)
    slot_t = slot_t.at[:, :N].set(slot.T).reshape(-1)
    gidx_flat = jnp.full((GTOT,), GPAD, jnp.int32)
    gidx_flat = gidx_flat.at[:VIEWS * NP].set(
        jnp.full((VIEWS, NP), GPAD, jnp.int32).at[:, :N].set(gidx.T).reshape(-1))
    gidx_rs = jnp.pad(gidx_flat.reshape(NWORK, 37, 128),
                      ((0, 0), (0, 11), (0, 0)), constant_values=GPAD)

    y_sep, st_sep = _mm_stats([(f3d_pad, 0, W_sep, None)], N)
    s_sep, o_sep = _finalize(st_sep, N, g_sep, b_sep)
    a_mat = _ew_affine_relu(y_sep, s_sep, o_sep, BM)
    a_ext = jnp.concatenate([a_mat, jnp.zeros((8, C), f32)], axis=0)

    init = jnp.full((SLOTS,), -1, jnp.int32)
    w32 = _winner_phase1(slot_t, init).reshape(NWORK, SLOTS)
    dense2d = _winner_reduce_gather(w32, a_ext)
    gact = _sc_gather(f2v_ext, gidx_rs)

    wv = W_vf1.reshape(VIEWS, C, C)
    y_vf1, st1 = _mm_stats(
        [(gact, 0, wv[0], None), (gact, NB, wv[1], None),
         (gact, 2 * NB, wv[2], None)], N)
    s1, o1 = _finalize(st1, N, g_vf1, b_vf1)
    y_vf2, st2 = _mm_stats([(y_vf1, 0, W_vf2, (s1, o1))], N)
    s2, o2 = _finalize(st2, N, g_vf2, b_vf2)
    wf = W_f3d.reshape(2, C, C)
    y_f3d, st3 = _mm_stats(
        [(f3d_pad, 0, wf[0], None), (y_vf2, 0, wf[1], (s2, o2))], N)
    s3, o3 = _finalize(st3, N, g_f3d, b_f3d)
    fused_3d = _ew_affine_relu(y_f3d, s3, o3, BM)[:N]

    cat = jnp.concatenate(
        [f2d_nhwc, dense2d.reshape(VIEWS * B, H, W, C)], axis=-1)
    xpad = jnp.pad(cat, ((0, 0), (1, 1), (1, 1), (0, 0)))
    xflat = jnp.pad(xpad.reshape(VIEWS * B, 66 * 66, 2 * C),
                    ((0, 0), (0, XL - 66 * 66), (0, 0)))
    wr = W_conv.transpose(2, 3, 1, 0).reshape(9 * 2 * C, C)
    y_cw, stc = _conv_stats(xflat, wr)
    sc_, oc_ = _finalize(stc, 6 * H * W, g_2d, b_2d)
    y2 = _ew_affine_relu(y_cw.reshape(6 * FL, C), sc_, oc_, FL)
    fused_2d = (y2.reshape(VIEWS * B, H, 66, C)[:, :, :64, :]
                .transpose(0, 3, 1, 2))
    return fused_3d, fused_2d


# speedup vs baseline: 6.3100x; 6.3071x over previous
import functools

import jax
import jax.numpy as jnp
from jax import lax
from jax.experimental import pallas as pl
from jax.experimental.pallas import tpu as pltpu
from jax.experimental.pallas import tpu_sc as plsc

VIEWS = 3
B = 2
C = 128
H = 64
W = 64
N = 50000
NP = 50176
SLOTS = VIEWS * B * H * W
NWORK = 32
CHUNK = NP // NWORK
SPT = SLOTS // NWORK
PAD_SLOT = 32000
GPAD = SLOTS
DENSE_PAD = NP
GTOT = 32 * 37 * 128
BM = 1792
NB = NP // BM

def _mesh():
    return plsc.VectorSubcoreMesh(core_axis_name="c", subcore_axis_name="s")


def _winner_phase1(slots, init):
    @functools.partial(
        pl.kernel, mesh=_mesh(),
        compiler_params=pltpu.CompilerParams(needs_layout_passes=False),
        out_type=jax.ShapeDtypeStruct((NWORK * SLOTS,), jnp.int32),
        scratch_types=[
            pltpu.VMEM((CHUNK,), jnp.int32),
            pltpu.VMEM((SLOTS,), jnp.int32),
        ],
    )
    def k(slots_hbm, init_hbm, out_hbm, chunk_v, win_v):
        wid = lax.axis_index("s") * 2 + lax.axis_index("c")
        pltpu.sync_copy(init_hbm, win_v)
        lanes = lax.iota(jnp.int32, 16)
        for v in range(VIEWS):
            pltpu.sync_copy(
                slots_hbm.at[pl.ds(v * NP + wid * CHUNK, CHUNK)], chunk_v)
            base_n = wid * CHUNK

            def body(i, _):
                s = chunk_v[pl.ds(i * 16, 16)]
                n = base_n + i * 16 + lanes
                valid = s < SLOTS
                _, last = plsc.scan_count(s, valid)
                plsc.store_scatter(win_v, [s], n, mask=last & valid)
                return 0

            lax.fori_loop(0, CHUNK // 16, body, 0)
        pltpu.sync_copy(win_v, out_hbm.at[pl.ds(wid * SLOTS, SLOTS)])

    return k(slots, init)


def _winner_reduce_gather(w32, a_ext):
    @functools.partial(
        pl.kernel, mesh=_mesh(),
        compiler_params=pltpu.CompilerParams(needs_layout_passes=False),
        out_type=jax.ShapeDtypeStruct((SLOTS, C), jnp.float32),
        scratch_types=[
            pltpu.VMEM((NWORK, SPT), jnp.int32),
            pltpu.VMEM((6, 128), jnp.int32),
            pltpu.VMEM((SPT, C), jnp.float32),
            pltpu.SemaphoreType.DMA,
        ],
    )
    def k(w_hbm, a_hbm, out_hbm, red_v, idx_v, rows_v, sem):
        wid = lax.axis_index("s") * 2 + lax.axis_index("c")
        base = wid * SPT
        pltpu.sync_copy(w_hbm.at[:, pl.ds(base, SPT)], red_v)
        for r in range(6):
            def body(c2, _):
                off = (r * 8 + c2) * 16
                m = red_v[0, pl.ds(off, 16)]
                for j in range(1, NWORK):
                    m = jnp.maximum(m, red_v[j, pl.ds(off, 16)])
                m = jnp.where(m < 0, DENSE_PAD, m)
                idx_v[r, pl.ds(c2 * 16, 16)] = m
                return 0

            lax.fori_loop(0, 8, body, 0)
        for r in range(6):
            pltpu.async_copy(a_hbm.at[idx_v.at[r]],
                             rows_v.at[pl.ds(r * 128, 128)], sem).wait()
        pltpu.sync_copy(rows_v, out_hbm.at[pl.ds(base, SPT)])

    return k(w32, a_ext)


def _sc_gather(tab, idx):
    @functools.partial(
        pl.kernel, mesh=_mesh(),
        compiler_params=pltpu.CompilerParams(needs_layout_passes=False),
        out_type=jax.ShapeDtypeStruct((GTOT, C), jnp.float32),
        scratch_types=[
            pltpu.VMEM((48, 128), jnp.int32),
            pltpu.VMEM((128, C), jnp.float32),
            pltpu.VMEM((128, C), jnp.float32),
            pltpu.SemaphoreType.DMA,
            pltpu.SemaphoreType.DMA,
        ],
    )
    def k(tab_hbm, idx_hbm, out_hbm, idx_v, buf0, buf1, sem0, sem1):
        wid = lax.axis_index("s") * 2 + lax.axis_index("c")
        pltpu.sync_copy(idx_hbm.at[wid], idx_v)
        bufs = (buf0, buf1)
        sems = (sem0, sem1)
        cps = [None, None]
        for j in range(37):
            p = j % 2
            cps[p] = pltpu.async_copy(tab_hbm.at[idx_v.at[j]], bufs[p],
                                      sems[p])
            if j > 0:
                q = (j - 1) % 2
                cps[q].wait()
                pltpu.sync_copy(
                    bufs[q],
                    out_hbm.at[pl.ds((wid * 37 + j - 1) * 128, 128)])
        cps[36 % 2].wait()
        pltpu.sync_copy(bufs[36 % 2],
                        out_hbm.at[pl.ds((wid * 37 + 36) * 128, 128)])

    return k(tab, idx)


def _mm_stats(terms, n_valid):
    nt = len(terms)

    def body(*refs):
        i = pl.program_id(0)
        xr = refs[0:nt]
        wr = refs[nt:2 * nt]
        pos = 2 * nt
        affr = []
        for t in range(nt):
            if terms[t][3] is not None:
                affr.append((refs[pos], refs[pos + 1]))
                pos += 2
            else:
                affr.append(None)
        y_ref, st_ref, acc_ref = refs[pos], refs[pos + 1], refs[pos + 2]
        y = jnp.zeros((BM, C), jnp.float32)
        for t in range(nt):
            xv = xr[t][...]
            if affr[t] is not None:
                s, b = affr[t]
                xv = jnp.maximum(xv * s[...] + b[...], 0.0)
            y = y + jnp.dot(xv, wr[t][...],
                            preferred_element_type=jnp.float32)
        y_ref[...] = y
        rows = i * BM + lax.broadcasted_iota(jnp.int32, (BM, 1), 0)
        ym = jnp.where(rows < n_valid, y, 0.0)

        @pl.when(i == 0)
        def _():
            acc_ref[...] = jnp.zeros_like(acc_ref)

        acc_ref[0, :] += jnp.sum(ym, axis=0)
        acc_ref[1, :] += jnp.sum(ym * ym, axis=0)

        @pl.when(i == NB - 1)
        def _():
            st_ref[...] = acc_ref[...]

    in_specs = []
    args = []
    for (x, off, w, aff) in terms:
        in_specs.append(pl.BlockSpec((BM, C), lambda i, o=off: (i + o, 0)))
        args.append(x)
    for (x, off, w, aff) in terms:
        in_specs.append(pl.BlockSpec((C, C), lambda i: (0, 0)))
        args.append(w)
    for (x, off, w, aff) in terms:
        if aff is not None:
            in_specs.append(pl.BlockSpec((1, C), lambda i: (0, 0)))
            in_specs.append(pl.BlockSpec((1, C), lambda i: (0, 0)))
            args.append(aff[0])
            args.append(aff[1])

    return pl.pallas_call(
        body,
        grid=(NB,),
        in_specs=in_specs,
        out_specs=[pl.BlockSpec((BM, C), lambda i: (i, 0)),
                   pl.BlockSpec((2, C), lambda i: (0, 0))],
        out_shape=[jax.ShapeDtypeStruct((NP, C), jnp.float32),
                   jax.ShapeDtypeStruct((2, C), jnp.float32)],
        scratch_shapes=[pltpu.VMEM((2, C), jnp.float32)],
    )(*args)


def _ew_affine_relu(x, s, b, bm):
    rows = x.shape[0]
    nb = rows // bm

    def body(x_ref, s_ref, b_ref, o_ref):
        o_ref[...] = jnp.maximum(x_ref[...] * s_ref[...] + b_ref[...], 0.0)

    return pl.pallas_call(
        body,
        grid=(nb,),
        in_specs=[pl.BlockSpec((bm, C), lambda i: (i, 0)),
                  pl.BlockSpec((1, C), lambda i: (0, 0)),
                  pl.BlockSpec((1, C), lambda i: (0, 0))],
        out_specs=pl.BlockSpec((bm, C), lambda i: (i, 0)),
        out_shape=jax.ShapeDtypeStruct((rows, C), jnp.float32),
    )(x, s, b)


FL = 4224
XL = 4360


def _conv_stats(xflat, wr):
    def body(x_ref, w_ref, y_ref, st_ref, acc_ref):
        i = pl.program_id(0)
        y = jnp.zeros((FL, C), jnp.float32)
        for dy in range(3):
            for dx in range(3):
                off = 66 * dy + dx
                t = dy * 3 + dx
                xs = x_ref[0, pl.ds(off, FL), :]
                y = y + jnp.dot(xs, w_ref[pl.ds(t * 256, 256), :],
                                preferred_element_type=jnp.float32)
        y_ref[0] = y
        col = lax.broadcasted_iota(jnp.int32, (FL, C), 0) % 66
        ym = jnp.where(col < 64, y, 0.0)

        @pl.when(i == 0)
        def _():
            acc_ref[...] = jnp.zeros_like(acc_ref)

        acc_ref[0, :] += jnp.sum(ym, axis=0)
        acc_ref[1, :] += jnp.sum(ym * ym, axis=0)

        @pl.when(i == 5)
        def _():
            st_ref[...] = acc_ref[...]

    return pl.pallas_call(
        body,
        grid=(6,),
        in_specs=[pl.BlockSpec((1, XL, 256), lambda i: (i, 0, 0)),
                  pl.BlockSpec((2304, C), lambda i: (0, 0))],
        out_specs=[pl.BlockSpec((1, FL, C), lambda i: (i, 0, 0)),
                   pl.BlockSpec((2, C), lambda i: (0, 0))],
        out_shape=[jax.ShapeDtypeStruct((6, FL, C), jnp.float32),
                   jax.ShapeDtypeStruct((2, C), jnp.float32)],
        scratch_shapes=[pltpu.VMEM((2, C), jnp.float32)],
    )(xflat, wr)


def _finalize(st, cnt, g, b):
    mean = st[0] / cnt
    var = st[1] / cnt - mean * mean
    scale = g / jnp.sqrt(var + 1e-5)
    bias = b - mean * scale
    return scale.reshape(1, C), bias.reshape(1, C)


def kernel(feat_2d_all, feat_3d_F, links, W_sep, g_sep, b_sep,
           W_vf1, g_vf1, b_vf1, W_vf2, g_vf2, b_vf2,
           W_f3d, g_f3d, b_f3d, W_conv, g_2d, b_2d):
    f32 = jnp.float32
    f2d_nhwc = feat_2d_all.reshape(VIEWS * B, C, H, W).transpose(0, 2, 3, 1)
    f2v_ext = jnp.concatenate(
        [f2d_nhwc.reshape(SLOTS, C), jnp.zeros((128, C), f32)], axis=0)
    f3d_pad = jnp.concatenate(
        [feat_3d_F, jnp.zeros((NP - N, C), f32)], axis=0)

    varr = jnp.arange(VIEWS, dtype=jnp.int32)[None, :]
    bi = links[:, 0, :]
    hi = links[:, 1, :]
    wi = links[:, 2, :]
    fl = links[:, 3, :]
    slot = ((varr * B + bi) * H + hi) * W + wi
    zrow = GPAD + (jnp.arange(N, dtype=jnp.int32)[:, None] % 128)
    gidx = jnp.where(fl == 1, slot, zrow)

    slot_t = jnp.full((VIEWS, NP), PAD_SLOT, jnp.int32)
    slot_t = slot_t.at[:, :N].set(slot.T).reshape(-1)
    zfill = GPAD + (jnp.arange(GTOT, dtype=jnp.int32) % 128)
    gidx_flat = zfill.at[:VIEWS * NP].set(
        jnp.where(
            jnp.zeros((VIEWS, NP), jnp.bool_).at[:, :N].set(True),
            jnp.zeros((VIEWS, NP), jnp.int32).at[:, :N].set(gidx.T),
            zfill[:VIEWS * NP].reshape(VIEWS, NP)).reshape(-1))
    gidx_rs = jnp.pad(gidx_flat.reshape(NWORK, 37, 128),
                      ((0, 0), (0, 11), (0, 0)), constant_values=GPAD)

    y_sep, st_sep = _mm_stats([(f3d_pad, 0, W_sep, None)], N)
    s_sep, o_sep = _finalize(st_sep, N, g_sep, b_sep)
    a_mat = _ew_affine_relu(y_sep, s_sep, o_sep, BM)
    a_ext = jnp.concatenate([a_mat, jnp.zeros((8, C), f32)], axis=0)

    init = jnp.full((SLOTS,), -1, jnp.int32)
    w32 = _winner_phase1(slot_t, init).reshape(NWORK, SLOTS)
    dense2d = _winner_reduce_gather(w32, a_ext)
    gact = _sc_gather(f2v_ext, gidx_rs)

    wv = W_vf1.reshape(VIEWS, C, C)
    y_vf1, st1 = _mm_stats(
        [(gact, 0, wv[0], None), (gact, NB, wv[1], None),
         (gact, 2 * NB, wv[2], None)], N)
    s1, o1 = _finalize(st1, N, g_vf1, b_vf1)
    y_vf2, st2 = _mm_stats([(y_vf1, 0, W_vf2, (s1, o1))], N)
    s2, o2 = _finalize(st2, N, g_vf2, b_vf2)
    wf = W_f3d.reshape(2, C, C)
    y_f3d, st3 = _mm_stats(
        [(f3d_pad, 0, wf[0], None), (y_vf2, 0, wf[1], (s2, o2))], N)
    s3, o3 = _finalize(st3, N, g_f3d, b_f3d)
    fused_3d = _ew_affine_relu(y_f3d, s3, o3, BM)[:N]

    cat = jnp.concatenate(
        [f2d_nhwc, dense2d.reshape(VIEWS * B, H, W, C)], axis=-1)
    xpad = jnp.pad(cat, ((0, 0), (1, 1), (1, 1), (0, 0)))
    xflat = jnp.pad(xpad.reshape(VIEWS * B, 66 * 66, 2 * C),
                    ((0, 0), (0, XL - 66 * 66), (0, 0)))
    wr = W_conv.transpose(2, 3, 1, 0).reshape(9 * 2 * C, C)
    y_cw, stc = _conv_stats(xflat, wr)
    sc_, oc_ = _finalize(stc, 6 * H * W, g_2d, b_2d)
    y2 = _ew_affine_relu(y_cw.reshape(6 * FL, C), sc_, oc_, FL)
    fused_2d = (y2.reshape(VIEWS * B, H, 66, C)[:, :, :64, :]
                .transpose(0, 3, 1, 2))
    return fused_3d, fused_2d


# speedup vs baseline: 7.2616x; 1.1508x over previous
import functools

import jax
import jax.numpy as jnp
from jax import lax
from jax.experimental import pallas as pl
from jax.experimental.pallas import tpu as pltpu
from jax.experimental.pallas import tpu_sc as plsc

VIEWS = 3
B = 2
C = 128
H = 64
W = 64
N = 50000
NP = 50176
SLOTS = VIEWS * B * H * W
NWORK = 32
CHUNK = NP // NWORK
SPT = SLOTS // NWORK
PAD_SLOT = 32000
GPAD = SLOTS
DENSE_PAD = NP
GTOT = 32 * 37 * 128
BM = 1792
NB = NP // BM

def _mesh():
    return plsc.VectorSubcoreMesh(core_axis_name="c", subcore_axis_name="s")


def _winner_phase1(slots, init):
    @functools.partial(
        pl.kernel, mesh=_mesh(),
        compiler_params=pltpu.CompilerParams(needs_layout_passes=False),
        out_type=jax.ShapeDtypeStruct((NWORK * SLOTS,), jnp.int32),
        scratch_types=[
            pltpu.VMEM((CHUNK,), jnp.int32),
            pltpu.VMEM((SLOTS,), jnp.int32),
        ],
    )
    def k(slots_hbm, init_hbm, out_hbm, chunk_v, win_v):
        wid = lax.axis_index("s") * 2 + lax.axis_index("c")
        pltpu.sync_copy(init_hbm, win_v)
        lanes = lax.iota(jnp.int32, 16)
        for v in range(VIEWS):
            pltpu.sync_copy(
                slots_hbm.at[pl.ds(v * NP + wid * CHUNK, CHUNK)], chunk_v)
            base_n = wid * CHUNK

            def body(i, _):
                s = chunk_v[pl.ds(i * 16, 16)]
                n = base_n + i * 16 + lanes
                valid = s < SLOTS
                _, last = plsc.scan_count(s, valid)
                plsc.store_scatter(win_v, [s], n, mask=last & valid)
                return 0

            lax.fori_loop(0, CHUNK // 16, body, 0)
        pltpu.sync_copy(win_v, out_hbm.at[pl.ds(wid * SLOTS, SLOTS)])

    return k(slots, init)


def _winner_reduce_gather(w32, a_ext):
    @functools.partial(
        pl.kernel, mesh=_mesh(),
        compiler_params=pltpu.CompilerParams(needs_layout_passes=False),
        out_type=jax.ShapeDtypeStruct((SLOTS, C), jnp.float32),
        scratch_types=[
            pltpu.VMEM((NWORK, SPT), jnp.int32),
            pltpu.VMEM((6, 128), jnp.int32),
            pltpu.VMEM((SPT, C), jnp.float32),
            pltpu.SemaphoreType.DMA,
        ],
    )
    def k(w_hbm, a_hbm, out_hbm, red_v, idx_v, rows_v, sem):
        wid = lax.axis_index("s") * 2 + lax.axis_index("c")
        base = wid * SPT
        pltpu.sync_copy(w_hbm.at[:, pl.ds(base, SPT)], red_v)
        for r in range(6):
            def body(c2, _):
                off = (r * 8 + c2) * 16
                m = red_v[0, pl.ds(off, 16)]
                for j in range(1, NWORK):
                    m = jnp.maximum(m, red_v[j, pl.ds(off, 16)])
                m = jnp.where(m < 0, DENSE_PAD, m)
                idx_v[r, pl.ds(c2 * 16, 16)] = m
                return 0

            lax.fori_loop(0, 8, body, 0)
        for r in range(6):
            pltpu.async_copy(a_hbm.at[idx_v.at[r]],
                             rows_v.at[pl.ds(r * 128, 128)], sem).wait()
        pltpu.sync_copy(rows_v, out_hbm.at[pl.ds(base, SPT)])

    return k(w32, a_ext)


def _sc_gather(tab, idx):
    @functools.partial(
        pl.kernel, mesh=_mesh(),
        compiler_params=pltpu.CompilerParams(needs_layout_passes=False),
        out_type=jax.ShapeDtypeStruct((GTOT, C), jnp.float32),
        scratch_types=[
            pltpu.VMEM((48, 128), jnp.int32),
            pltpu.VMEM((128, C), jnp.float32),
            pltpu.VMEM((128, C), jnp.float32),
            pltpu.SemaphoreType.DMA,
            pltpu.SemaphoreType.DMA,
        ],
    )
    def k(tab_hbm, idx_hbm, out_hbm, idx_v, buf0, buf1, sem0, sem1):
        wid = lax.axis_index("s") * 2 + lax.axis_index("c")
        pltpu.sync_copy(idx_hbm.at[wid], idx_v)
        bufs = (buf0, buf1)
        sems = (sem0, sem1)
        cps = [None, None]
        for j in range(37):
            p = j % 2
            cps[p] = pltpu.async_copy(tab_hbm.at[idx_v.at[j]], bufs[p],
                                      sems[p])
            if j > 0:
                q = (j - 1) % 2
                cps[q].wait()
                pltpu.sync_copy(
                    bufs[q],
                    out_hbm.at[pl.ds((wid * 37 + j - 1) * 128, 128)])
        cps[36 % 2].wait()
        pltpu.sync_copy(bufs[36 % 2],
                        out_hbm.at[pl.ds((wid * 37 + 36) * 128, 128)])

    return k(tab, idx)


def _mm_stats(terms, n_valid):
    nt = len(terms)

    def body(*refs):
        i = pl.program_id(0)
        xr = refs[0:nt]
        wr = refs[nt:2 * nt]
        pos = 2 * nt
        affr = []
        for t in range(nt):
            if terms[t][3] is not None:
                affr.append((refs[pos], refs[pos + 1]))
                pos += 2
            else:
                affr.append(None)
        y_ref, st_ref, acc_ref = refs[pos], refs[pos + 1], refs[pos + 2]
        y = jnp.zeros((BM, C), jnp.float32)
        for t in range(nt):
            xv = xr[t][...]
            if affr[t] is not None:
                s, b = affr[t]
                xv = jnp.maximum(xv * s[...] + b[...], 0.0)
            y = y + jnp.dot(xv, wr[t][...],
                            preferred_element_type=jnp.float32)
        y_ref[...] = y
        rows = i * BM + lax.broadcasted_iota(jnp.int32, (BM, 1), 0)
        ym = jnp.where(rows < n_valid, y, 0.0)

        @pl.when(i == 0)
        def _():
            acc_ref[...] = jnp.zeros_like(acc_ref)

        acc_ref[0, :] += jnp.sum(ym, axis=0)
        acc_ref[1, :] += jnp.sum(ym * ym, axis=0)

        @pl.when(i == NB - 1)
        def _():
            st_ref[...] = acc_ref[...]

    in_specs = []
    args = []
    for (x, off, w, aff) in terms:
        in_specs.append(pl.BlockSpec((BM, C), lambda i, o=off: (i + o, 0)))
        args.append(x)
    for (x, off, w, aff) in terms:
        in_specs.append(pl.BlockSpec((C, C), lambda i: (0, 0)))
        args.append(w)
    for (x, off, w, aff) in terms:
        if aff is not None:
            in_specs.append(pl.BlockSpec((1, C), lambda i: (0, 0)))
            in_specs.append(pl.BlockSpec((1, C), lambda i: (0, 0)))
            args.append(aff[0])
            args.append(aff[1])

    return pl.pallas_call(
        body,
        grid=(NB,),
        in_specs=in_specs,
        out_specs=[pl.BlockSpec((BM, C), lambda i: (i, 0)),
                   pl.BlockSpec((2, C), lambda i: (0, 0))],
        out_shape=[jax.ShapeDtypeStruct((NP, C), jnp.float32),
                   jax.ShapeDtypeStruct((2, C), jnp.float32)],
        scratch_shapes=[pltpu.VMEM((2, C), jnp.float32)],
    )(*args)


def _ew_affine_relu(x, s, b, bm):
    rows = x.shape[0]
    nb = rows // bm

    def body(x_ref, s_ref, b_ref, o_ref):
        o_ref[...] = jnp.maximum(x_ref[...] * s_ref[...] + b_ref[...], 0.0)

    return pl.pallas_call(
        body,
        grid=(nb,),
        in_specs=[pl.BlockSpec((bm, C), lambda i: (i, 0)),
                  pl.BlockSpec((1, C), lambda i: (0, 0)),
                  pl.BlockSpec((1, C), lambda i: (0, 0))],
        out_specs=pl.BlockSpec((bm, C), lambda i: (i, 0)),
        out_shape=jax.ShapeDtypeStruct((rows, C), jnp.float32),
    )(x, s, b)


FLW = 72
FL = 64 * FLW
XL = 4768


def _conv_stats(x2d, xden, wr):
    def body(x2_ref, xd_ref, w_ref, y_ref, st_ref, acc_ref, xs_ref):
        i = pl.program_id(0)
        xs_ref[...] = jnp.zeros((XL, 256), jnp.float32)

        def cp(y_row, _):
            dst = (y_row + 1) * FLW + 8
            xs_ref[pl.ds(dst, W), 0:C] = x2_ref[0, pl.ds(y_row * W, W), :]
            xs_ref[pl.ds(dst, W), C:2 * C] = xd_ref[0, pl.ds(y_row * W, W), :]
            return 0

        lax.fori_loop(0, H, cp, 0)
        y = jnp.zeros((FL, C), jnp.float32)
        for dy in range(3):
            for dx in range(3):
                off = FLW * dy + dx + 7
                t = dy * 3 + dx
                xs = xs_ref[pl.ds(off, FL), :]
                y = y + jnp.dot(xs, w_ref[pl.ds(t * 256, 256), :],
                                preferred_element_type=jnp.float32)
        y_ref[0] = y
        col = lax.broadcasted_iota(jnp.int32, (FL, C), 0) % FLW
        ym = jnp.where(col < 64, y, 0.0)

        @pl.when(i == 0)
        def _():
            acc_ref[...] = jnp.zeros_like(acc_ref)

        acc_ref[0, :] += jnp.sum(ym, axis=0)
        acc_ref[1, :] += jnp.sum(ym * ym, axis=0)

        @pl.when(i == 5)
        def _():
            st_ref[...] = acc_ref[...]

    return pl.pallas_call(
        body,
        grid=(6,),
        in_specs=[pl.BlockSpec((1, H * W, C), lambda i: (i, 0, 0)),
                  pl.BlockSpec((1, H * W, C), lambda i: (i, 0, 0)),
                  pl.BlockSpec((2304, C), lambda i: (0, 0))],
        out_specs=[pl.BlockSpec((1, FL, C), lambda i: (i, 0, 0)),
                   pl.BlockSpec((2, C), lambda i: (0, 0))],
        out_shape=[jax.ShapeDtypeStruct((6, FL, C), jnp.float32),
                   jax.ShapeDtypeStruct((2, C), jnp.float32)],
        scratch_shapes=[pltpu.VMEM((2, C), jnp.float32),
                        pltpu.VMEM((XL, 256), jnp.float32)],
    )(x2d, xden, wr)


def _finalize(st, cnt, g, b):
    mean = st[0] / cnt
    var = st[1] / cnt - mean * mean
    scale = g / jnp.sqrt(var + 1e-5)
    bias = b - mean * scale
    return scale.reshape(1, C), bias.reshape(1, C)


def kernel(feat_2d_all, feat_3d_F, links, W_sep, g_sep, b_sep,
           W_vf1, g_vf1, b_vf1, W_vf2, g_vf2, b_vf2,
           W_f3d, g_f3d, b_f3d, W_conv, g_2d, b_2d):
    f32 = jnp.float32
    f2d_nhwc = feat_2d_all.reshape(VIEWS * B, C, H, W).transpose(0, 2, 3, 1)
    f2v_ext = jnp.concatenate(
        [f2d_nhwc.reshape(SLOTS, C), jnp.zeros((128, C), f32)], axis=0)
    f3d_pad = jnp.concatenate(
        [feat_3d_F, jnp.zeros((NP - N, C), f32)], axis=0)

    varr = jnp.arange(VIEWS, dtype=jnp.int32)[None, :]
    bi = links[:, 0, :]
    hi = links[:, 1, :]
    wi = links[:, 2, :]
    fl = links[:, 3, :]
    slot = ((varr * B + bi) * H + hi) * W + wi
    zrow = GPAD + (jnp.arange(N, dtype=jnp.int32)[:, None] % 128)
    gidx = jnp.where(fl == 1, slot, zrow)

    slot_t = jnp.full((VIEWS, NP), PAD_SLOT, jnp.int32)
    slot_t = slot_t.at[:, :N].set(slot.T).reshape(-1)
    zfill = GPAD + (jnp.arange(GTOT, dtype=jnp.int32) % 128)
    gidx_flat = zfill.at[:VIEWS * NP].set(
        jnp.where(
            jnp.zeros((VIEWS, NP), jnp.bool_).at[:, :N].set(True),
            jnp.zeros((VIEWS, NP), jnp.int32).at[:, :N].set(gidx.T),
            zfill[:VIEWS * NP].reshape(VIEWS, NP)).reshape(-1))
    gidx_rs = jnp.pad(gidx_flat.reshape(NWORK, 37, 128),
                      ((0, 0), (0, 11), (0, 0)), constant_values=GPAD)

    y_sep, st_sep = _mm_stats([(f3d_pad, 0, W_sep, None)], N)
    s_sep, o_sep = _finalize(st_sep, N, g_sep, b_sep)
    a_mat = _ew_affine_relu(y_sep, s_sep, o_sep, BM)
    a_ext = jnp.concatenate([a_mat, jnp.zeros((8, C), f32)], axis=0)

    init = jnp.full((SLOTS,), -1, jnp.int32)
    w32 = _winner_phase1(slot_t, init).reshape(NWORK, SLOTS)
    dense2d = _winner_reduce_gather(w32, a_ext)
    gact = _sc_gather(f2v_ext, gidx_rs)

    wv = W_vf1.reshape(VIEWS, C, C)
    y_vf1, st1 = _mm_stats(
        [(gact, 0, wv[0], None), (gact, NB, wv[1], None),
         (gact, 2 * NB, wv[2], None)], N)
    s1, o1 = _finalize(st1, N, g_vf1, b_vf1)
    y_vf2, st2 = _mm_stats([(y_vf1, 0, W_vf2, (s1, o1))], N)
    s2, o2 = _finalize(st2, N, g_vf2, b_vf2)
    wf = W_f3d.reshape(2, C, C)
    y_f3d, st3 = _mm_stats(
        [(f3d_pad, 0, wf[0], None), (y_vf2, 0, wf[1], (s2, o2))], N)
    s3, o3 = _finalize(st3, N, g_f3d, b_f3d)
    fused_3d = _ew_affine_relu(y_f3d, s3, o3, BM)[:N]

    wr = W_conv.transpose(2, 3, 1, 0).reshape(9 * 2 * C, C)
    y_cw, stc = _conv_stats(f2d_nhwc.reshape(VIEWS * B, H * W, C),
                            dense2d.reshape(VIEWS * B, H * W, C), wr)
    sc_, oc_ = _finalize(stc, 6 * H * W, g_2d, b_2d)
    y2 = _ew_affine_relu(y_cw.reshape(6 * FL, C), sc_, oc_, FL)
    fused_2d = (y2.reshape(VIEWS * B, H, FLW, C)[:, :, :64, :]
                .transpose(0, 3, 1, 2))
    return fused_3d, fused_2d


# speedup vs baseline: 8.0112x; 1.1032x over previous
import functools

import jax
import jax.numpy as jnp
from jax import lax
from jax.experimental import pallas as pl
from jax.experimental.pallas import tpu as pltpu
from jax.experimental.pallas import tpu_sc as plsc

VIEWS = 3
B = 2
C = 128
H = 64
W = 64
N = 50000
NP = 50176
SLOTS = VIEWS * B * H * W
NWORK = 32
CHUNK = NP // NWORK
SPT = SLOTS // NWORK
PAD_SLOT = 32000
GPAD = SLOTS
DENSE_PAD = NP
NP2 = NP + 128
GTOT = 32 * 37 * 128
BM = 1792
NB = NP // BM

def _mesh():
    return plsc.VectorSubcoreMesh(core_axis_name="c", subcore_axis_name="s")


def _winner_phase1(slots, init):
    @functools.partial(
        pl.kernel, mesh=_mesh(),
        compiler_params=pltpu.CompilerParams(needs_layout_passes=False),
        out_type=jax.ShapeDtypeStruct((NWORK * SLOTS,), jnp.int32),
        scratch_types=[
            pltpu.VMEM((CHUNK,), jnp.int32),
            pltpu.VMEM((SLOTS,), jnp.int32),
        ],
    )
    def k(slots_hbm, init_hbm, out_hbm, chunk_v, win_v):
        wid = lax.axis_index("s") * 2 + lax.axis_index("c")
        pltpu.sync_copy(init_hbm, win_v)
        lanes = lax.iota(jnp.int32, 16)
        for v in range(VIEWS):
            pltpu.sync_copy(
                slots_hbm.at[pl.ds(v * NP + wid * CHUNK, CHUNK)], chunk_v)
            base_n = wid * CHUNK

            def body(i, _):
                s = chunk_v[pl.ds(i * 16, 16)]
                n = base_n + i * 16 + lanes
                valid = s < SLOTS
                _, last = plsc.scan_count(s, valid)
                plsc.store_scatter(win_v, [s], n, mask=last & valid)
                return 0

            lax.fori_loop(0, CHUNK // 16, body, 0)
        pltpu.sync_copy(win_v, out_hbm.at[pl.ds(wid * SLOTS, SLOTS)])

    return k(slots, init)


def _winner_reduce_gather(w32, a_ext):
    @functools.partial(
        pl.kernel, mesh=_mesh(),
        compiler_params=pltpu.CompilerParams(needs_layout_passes=False),
        out_type=jax.ShapeDtypeStruct((SLOTS, C), jnp.float32),
        scratch_types=[
            pltpu.VMEM((NWORK, SPT), jnp.int32),
            pltpu.VMEM((6, 128), jnp.int32),
            pltpu.VMEM((SPT, C), jnp.float32),
            pltpu.SemaphoreType.DMA,
        ],
    )
    def k(w_hbm, a_hbm, out_hbm, red_v, idx_v, rows_v, sem):
        wid = lax.axis_index("s") * 2 + lax.axis_index("c")
        base = wid * SPT
        lanes = lax.iota(jnp.int32, 16)
        pltpu.sync_copy(w_hbm.at[:, pl.ds(base, SPT)], red_v)
        for r in range(6):
            def body(c2, _):
                off = (r * 8 + c2) * 16
                m = red_v[0, pl.ds(off, 16)]
                for j in range(1, NWORK):
                    m = jnp.maximum(m, red_v[j, pl.ds(off, 16)])
                m = jnp.where(m < 0, DENSE_PAD + lanes, m)
                idx_v[r, pl.ds(c2 * 16, 16)] = m
                return 0

            lax.fori_loop(0, 8, body, 0)
        for r in range(6):
            pltpu.async_copy(a_hbm.at[idx_v.at[r]],
                             rows_v.at[pl.ds(r * 128, 128)], sem).wait()
        pltpu.sync_copy(rows_v, out_hbm.at[pl.ds(base, SPT)])

    return k(w32, a_ext)


def _sc_gather(tab, idx):
    @functools.partial(
        pl.kernel, mesh=_mesh(),
        compiler_params=pltpu.CompilerParams(needs_layout_passes=False),
        out_type=jax.ShapeDtypeStruct((GTOT, C), jnp.float32),
        scratch_types=[
            pltpu.VMEM((48, 128), jnp.int32),
            pltpu.VMEM((128, C), jnp.float32),
            pltpu.VMEM((128, C), jnp.float32),
            pltpu.SemaphoreType.DMA,
            pltpu.SemaphoreType.DMA,
        ],
    )
    def k(tab_hbm, idx_hbm, out_hbm, idx_v, buf0, buf1, sem0, sem1):
        wid = lax.axis_index("s") * 2 + lax.axis_index("c")
        pltpu.sync_copy(idx_hbm.at[wid], idx_v)
        bufs = (buf0, buf1)
        sems = (sem0, sem1)
        cps = [None, None]
        for j in range(37):
            p = j % 2
            cps[p] = pltpu.async_copy(tab_hbm.at[idx_v.at[j]], bufs[p],
                                      sems[p])
            if j > 0:
                q = (j - 1) % 2
                cps[q].wait()
                pltpu.sync_copy(
                    bufs[q],
                    out_hbm.at[pl.ds((wid * 37 + j - 1) * 128, 128)])
        cps[36 % 2].wait()
        pltpu.sync_copy(bufs[36 % 2],
                        out_hbm.at[pl.ds((wid * 37 + 36) * 128, 128)])

    return k(tab, idx)


def _mm_stats(terms, n_valid):
    nt = len(terms)

    def body(*refs):
        i = pl.program_id(0)
        xr = refs[0:nt]
        wr = refs[nt:2 * nt]
        pos = 2 * nt
        affr = []
        for t in range(nt):
            if terms[t][3] is not None:
                affr.append((refs[pos], refs[pos + 1]))
                pos += 2
            else:
                affr.append(None)
        y_ref, st_ref, acc_ref = refs[pos], refs[pos + 1], refs[pos + 2]
        y = jnp.zeros((BM, C), jnp.float32)
        for t in range(nt):
            xv = xr[t][...]
            if affr[t] is not None:
                s, b = affr[t]
                xv = jnp.maximum(xv * s[...] + b[...], 0.0)
            y = y + jnp.dot(xv, wr[t][...],
                            preferred_element_type=jnp.float32)
        y_ref[...] = y
        rows = i * BM + lax.broadcasted_iota(jnp.int32, (BM, 1), 0)
        ym = jnp.where(rows < n_valid, y, 0.0)

        @pl.when(i == 0)
        def _():
            acc_ref[...] = jnp.zeros_like(acc_ref)

        acc_ref[0, :] += jnp.sum(ym, axis=0)
        acc_ref[1, :] += jnp.sum(ym * ym, axis=0)

        @pl.when(i == NB - 1)
        def _():
            st_ref[...] = acc_ref[...]

    in_specs = []
    args = []
    for (x, off, w, aff) in terms:
        in_specs.append(pl.BlockSpec((BM, C), lambda i, o=off: (i + o, 0)))
        args.append(x)
    for (x, off, w, aff) in terms:
        in_specs.append(pl.BlockSpec((C, C), lambda i: (0, 0)))
        args.append(w)
    for (x, off, w, aff) in terms:
        if aff is not None:
            in_specs.append(pl.BlockSpec((1, C), lambda i: (0, 0)))
            in_specs.append(pl.BlockSpec((1, C), lambda i: (0, 0)))
            args.append(aff[0])
            args.append(aff[1])

    return pl.pallas_call(
        body,
        grid=(NB,),
        in_specs=in_specs,
        out_specs=[pl.BlockSpec((BM, C), lambda i: (i, 0)),
                   pl.BlockSpec((2, C), lambda i: (0, 0))],
        out_shape=[jax.ShapeDtypeStruct((NP, C), jnp.float32),
                   jax.ShapeDtypeStruct((2, C), jnp.float32)],
        scratch_shapes=[pltpu.VMEM((2, C), jnp.float32)],
    )(*args)


def _affine_relu_ext(x, s, b):
    bm = NP2 // 16

    def body(x_ref, s_ref, b_ref, o_ref):
        i = pl.program_id(0)
        rows = i * bm + lax.broadcasted_iota(jnp.int32, (bm, 1), 0)
        y = jnp.maximum(x_ref[...] * s_ref[...] + b_ref[...], 0.0)
        o_ref[...] = jnp.where(rows < NP, y, 0.0)

    return pl.pallas_call(
        body,
        grid=(16,),
        in_specs=[pl.BlockSpec((bm, C), lambda i: (i, 0)),
                  pl.BlockSpec((1, C), lambda i: (0, 0)),
                  pl.BlockSpec((1, C), lambda i: (0, 0))],
        out_specs=pl.BlockSpec((bm, C), lambda i: (i, 0)),
        out_shape=jax.ShapeDtypeStruct((NP2, C), jnp.float32),
    )(x, s, b)


def _affine_relu_head(x, s, b):
    bm = N // 5

    def body(x_ref, s_ref, b_ref, o_ref):
        o_ref[...] = jnp.maximum(x_ref[...] * s_ref[...] + b_ref[...], 0.0)

    return pl.pallas_call(
        body,
        grid=(5,),
        in_specs=[pl.BlockSpec((bm, C), lambda i: (i, 0)),
                  pl.BlockSpec((1, C), lambda i: (0, 0)),
                  pl.BlockSpec((1, C), lambda i: (0, 0))],
        out_specs=pl.BlockSpec((bm, C), lambda i: (i, 0)),
        out_shape=jax.ShapeDtypeStruct((N, C), jnp.float32),
    )(x, s, b)


def _ew_affine_relu(x, s, b, bm):
    rows = x.shape[0]
    nb = rows // bm

    def body(x_ref, s_ref, b_ref, o_ref):
        o_ref[...] = jnp.maximum(x_ref[...] * s_ref[...] + b_ref[...], 0.0)

    return pl.pallas_call(
        body,
        grid=(nb,),
        in_specs=[pl.BlockSpec((bm, C), lambda i: (i, 0)),
                  pl.BlockSpec((1, C), lambda i: (0, 0)),
                  pl.BlockSpec((1, C), lambda i: (0, 0))],
        out_specs=pl.BlockSpec((bm, C), lambda i: (i, 0)),
        out_shape=jax.ShapeDtypeStruct((rows, C), jnp.float32),
    )(x, s, b)


FLW = 72
FL = 64 * FLW
XL = 4768


def _conv_stats(x2d, xden, wr):
    def body(x2_ref, xd_ref, w_ref, y_ref, st_ref, acc_ref, xs_ref):
        i = pl.program_id(0)
        xs_ref[...] = jnp.zeros((XL, 256), jnp.float32)

        def cp(y_row, _):
            dst = (y_row + 1) * FLW + 8
            xs_ref[pl.ds(dst, W), 0:C] = x2_ref[0, pl.ds(y_row * W, W), :]
            xs_ref[pl.ds(dst, W), C:2 * C] = xd_ref[0, pl.ds(y_row * W, W), :]
            return 0

        lax.fori_loop(0, H, cp, 0)
        y = jnp.zeros((FL, C), jnp.float32)
        for dy in range(3):
            for dx in range(3):
                off = FLW * dy + dx + 7
                t = dy * 3 + dx
                xs = xs_ref[pl.ds(off, FL), :]
                y = y + jnp.dot(xs, w_ref[pl.ds(t * 256, 256), :],
                                preferred_element_type=jnp.float32)
        y_ref[0] = y
        col = lax.broadcasted_iota(jnp.int32, (FL, C), 0) % FLW
        ym = jnp.where(col < 64, y, 0.0)

        @pl.when(i == 0)
        def _():
            acc_ref[...] = jnp.zeros_like(acc_ref)

        acc_ref[0, :] += jnp.sum(ym, axis=0)
        acc_ref[1, :] += jnp.sum(ym * ym, axis=0)

        @pl.when(i == 5)
        def _():
            st_ref[...] = acc_ref[...]

    return pl.pallas_call(
        body,
        grid=(6,),
        in_specs=[pl.BlockSpec((1, H * W, C), lambda i: (i, 0, 0)),
                  pl.BlockSpec((1, H * W, C), lambda i: (i, 0, 0)),
                  pl.BlockSpec((2304, C), lambda i: (0, 0))],
        out_specs=[pl.BlockSpec((1, FL, C), lambda i: (i, 0, 0)),
                   pl.BlockSpec((2, C), lambda i: (0, 0))],
        out_shape=[jax.ShapeDtypeStruct((6, FL, C), jnp.float32),
                   jax.ShapeDtypeStruct((2, C), jnp.float32)],
        scratch_shapes=[pltpu.VMEM((2, C), jnp.float32),
                        pltpu.VMEM((XL, 256), jnp.float32)],
    )(x2d, xden, wr)


def _finalize(st, cnt, g, b):
    mean = st[0] / cnt
    var = st[1] / cnt - mean * mean
    scale = g / jnp.sqrt(var + 1e-5)
    bias = b - mean * scale
    return scale.reshape(1, C), bias.reshape(1, C)


def kernel(feat_2d_all, feat_3d_F, links, W_sep, g_sep, b_sep,
           W_vf1, g_vf1, b_vf1, W_vf2, g_vf2, b_vf2,
           W_f3d, g_f3d, b_f3d, W_conv, g_2d, b_2d):
    f32 = jnp.float32
    f2d_nhwc = feat_2d_all.reshape(VIEWS * B, C, H, W).transpose(0, 2, 3, 1)
    f2v_ext = jnp.concatenate(
        [f2d_nhwc.reshape(SLOTS, C), jnp.zeros((128, C), f32)], axis=0)
    f3d_pad = jnp.concatenate(
        [feat_3d_F, jnp.zeros((NP - N, C), f32)], axis=0)

    varr = jnp.arange(VIEWS, dtype=jnp.int32)[None, :]
    bi = links[:, 0, :]
    hi = links[:, 1, :]
    wi = links[:, 2, :]
    fl = links[:, 3, :]
    slot = ((varr * B + bi) * H + hi) * W + wi
    zrow = GPAD + (jnp.arange(N, dtype=jnp.int32)[:, None] % 128)
    gidx = jnp.where(fl == 1, slot, zrow)

    slot_t = jnp.full((VIEWS, NP), PAD_SLOT, jnp.int32)
    slot_t = slot_t.at[:, :N].set(slot.T).reshape(-1)
    zfill = GPAD + (jnp.arange(GTOT, dtype=jnp.int32) % 128)
    gidx_flat = zfill.at[:VIEWS * NP].set(
        jnp.where(
            jnp.zeros((VIEWS, NP), jnp.bool_).at[:, :N].set(True),
            jnp.zeros((VIEWS, NP), jnp.int32).at[:, :N].set(gidx.T),
            zfill[:VIEWS * NP].reshape(VIEWS, NP)).reshape(-1))
    gidx_rs = jnp.pad(gidx_flat.reshape(NWORK, 37, 128),
                      ((0, 0), (0, 11), (0, 0)), constant_values=GPAD)

    y_sep, st_sep = _mm_stats([(f3d_pad, 0, W_sep, None)], N)
    s_sep, o_sep = _finalize(st_sep, N, g_sep, b_sep)
    a_ext = _affine_relu_ext(y_sep, s_sep, o_sep)

    init = jnp.full((SLOTS,), -1, jnp.int32)
    w32 = _winner_phase1(slot_t, init).reshape(NWORK, SLOTS)
    dense2d = _winner_reduce_gather(w32, a_ext)
    gact = _sc_gather(f2v_ext, gidx_rs)

    wv = W_vf1.reshape(VIEWS, C, C)
    y_vf1, st1 = _mm_stats(
        [(gact, 0, wv[0], None), (gact, NB, wv[1], None),
         (gact, 2 * NB, wv[2], None)], N)
    s1, o1 = _finalize(st1, N, g_vf1, b_vf1)
    y_vf2, st2 = _mm_stats([(y_vf1, 0, W_vf2, (s1, o1))], N)
    s2, o2 = _finalize(st2, N, g_vf2, b_vf2)
    wf = W_f3d.reshape(2, C, C)
    y_f3d, st3 = _mm_stats(
        [(f3d_pad, 0, wf[0], None), (y_vf2, 0, wf[1], (s2, o2))], N)
    s3, o3 = _finalize(st3, N, g_f3d, b_f3d)
    fused_3d = _affine_relu_head(y_f3d, s3, o3)

    wr = W_conv.transpose(2, 3, 1, 0).reshape(9 * 2 * C, C)
    y_cw, stc = _conv_stats(f2d_nhwc.reshape(VIEWS * B, H * W, C),
                            dense2d.reshape(VIEWS * B, H * W, C), wr)
    sc_, oc_ = _finalize(stc, 6 * H * W, g_2d, b_2d)
    y2 = _ew_affine_relu(y_cw.reshape(6 * FL, C), sc_, oc_, FL)
    fused_2d = (y2.reshape(VIEWS * B, H, FLW, C)[:, :, :64, :]
                .transpose(0, 3, 1, 2))
    return fused_3d, fused_2d


# speedup vs baseline: 8.0444x; 1.0041x over previous
import functools

import jax
import jax.numpy as jnp
from jax import lax
from jax.experimental import pallas as pl
from jax.experimental.pallas import tpu as pltpu
from jax.experimental.pallas import tpu_sc as plsc

VIEWS = 3
B = 2
C = 128
H = 64
W = 64
N = 50000
NP = 50176
SLOTS = VIEWS * B * H * W
NWORK = 32
CHUNK = NP // NWORK
SPT = SLOTS // NWORK
PAD_SLOT = 32000
GPAD = SLOTS
DENSE_PAD = NP
NP2 = NP + 128
GTOT = 32 * 37 * 128
BM = 1792
NB = NP // BM

def _mesh():
    return plsc.VectorSubcoreMesh(core_axis_name="c", subcore_axis_name="s")


def _winner_phase1(slots, init):
    @functools.partial(
        pl.kernel, mesh=_mesh(),
        compiler_params=pltpu.CompilerParams(needs_layout_passes=False),
        out_type=jax.ShapeDtypeStruct((NWORK * SLOTS,), jnp.int32),
        scratch_types=[
            pltpu.VMEM((CHUNK,), jnp.int32),
            pltpu.VMEM((SLOTS,), jnp.int32),
        ],
    )
    def k(slots_hbm, init_hbm, out_hbm, chunk_v, win_v):
        wid = lax.axis_index("s") * 2 + lax.axis_index("c")
        pltpu.sync_copy(init_hbm, win_v)
        lanes = lax.iota(jnp.int32, 16)
        for v in range(VIEWS):
            pltpu.sync_copy(
                slots_hbm.at[pl.ds(v * NP + wid * CHUNK, CHUNK)], chunk_v)
            base_n = wid * CHUNK

            def body(i, _):
                s = chunk_v[pl.ds(i * 16, 16)]
                n = base_n + i * 16 + lanes
                valid = s < SLOTS
                _, last = plsc.scan_count(s, valid)
                plsc.store_scatter(win_v, [s], n, mask=last & valid)
                return 0

            lax.fori_loop(0, CHUNK // 16, body, 0)
        pltpu.sync_copy(win_v, out_hbm.at[pl.ds(wid * SLOTS, SLOTS)])

    return k(slots, init)


def _winner_reduce_gather(w32, a_ext):
    @functools.partial(
        pl.kernel, mesh=_mesh(),
        compiler_params=pltpu.CompilerParams(needs_layout_passes=False),
        out_type=jax.ShapeDtypeStruct((SLOTS, C), jnp.float32),
        scratch_types=[
            pltpu.VMEM((NWORK, SPT), jnp.int32),
            pltpu.VMEM((6, 128), jnp.int32),
            pltpu.VMEM((SPT, C), jnp.float32),
            pltpu.SemaphoreType.DMA,
        ],
    )
    def k(w_hbm, a_hbm, out_hbm, red_v, idx_v, rows_v, sem):
        wid = lax.axis_index("s") * 2 + lax.axis_index("c")
        base = wid * SPT
        lanes = lax.iota(jnp.int32, 16)
        pltpu.sync_copy(w_hbm.at[:, pl.ds(base, SPT)], red_v)
        for r in range(6):
            def body(c2, _):
                off = (r * 8 + c2) * 16
                m = red_v[0, pl.ds(off, 16)]
                for j in range(1, NWORK):
                    m = jnp.maximum(m, red_v[j, pl.ds(off, 16)])
                m = jnp.where(m < 0, DENSE_PAD + lanes, m)
                idx_v[r, pl.ds(c2 * 16, 16)] = m
                return 0

            lax.fori_loop(0, 8, body, 0)
        for r in range(6):
            pltpu.async_copy(a_hbm.at[idx_v.at[r]],
                             rows_v.at[pl.ds(r * 128, 128)], sem).wait()
        pltpu.sync_copy(rows_v, out_hbm.at[pl.ds(base, SPT)])

    return k(w32, a_ext)


def _sc_gather(tab, idx):
    @functools.partial(
        pl.kernel, mesh=_mesh(),
        compiler_params=pltpu.CompilerParams(needs_layout_passes=False),
        out_type=jax.ShapeDtypeStruct((GTOT, C), jnp.float32),
        scratch_types=[
            pltpu.VMEM((48, 128), jnp.int32),
            pltpu.VMEM((128, C), jnp.float32),
            pltpu.VMEM((128, C), jnp.float32),
            pltpu.VMEM((128, C), jnp.float32),
            pltpu.VMEM((128, C), jnp.float32),
            pltpu.SemaphoreType.DMA,
            pltpu.SemaphoreType.DMA,
            pltpu.SemaphoreType.DMA,
            pltpu.SemaphoreType.DMA,
            pltpu.SemaphoreType.DMA,
            pltpu.SemaphoreType.DMA,
            pltpu.SemaphoreType.DMA,
            pltpu.SemaphoreType.DMA,
        ],
    )
    def k(tab_hbm, idx_hbm, out_hbm, idx_v, b0, b1, b2, b3,
          g0, g1, g2, g3, w0, w1, w2, w3):
        wid = lax.axis_index("s") * 2 + lax.axis_index("c")
        pltpu.sync_copy(idx_hbm.at[wid], idx_v)
        bufs = (b0, b1, b2, b3)
        gsems = (g0, g1, g2, g3)
        wsems = (w0, w1, w2, w3)
        cg = [None] * 4
        cw = [None] * 4
        for j in range(37):
            p = j % 4
            if j >= 4:
                cw[p].wait()
            cg[p] = pltpu.async_copy(tab_hbm.at[idx_v.at[j]], bufs[p],
                                     gsems[p])
            if j > 0:
                q = (j - 1) % 4
                cg[q].wait()
                cw[q] = pltpu.async_copy(
                    bufs[q],
                    out_hbm.at[pl.ds((wid * 37 + j - 1) * 128, 128)],
                    wsems[q])
        cg[36 % 4].wait()
        pltpu.sync_copy(bufs[36 % 4],
                        out_hbm.at[pl.ds((wid * 37 + 36) * 128, 128)])
        for q in (33 % 4, 34 % 4, 35 % 4):
            cw[q].wait()

    return k(tab, idx)


def _mm_stats(terms, n_valid):
    nt = len(terms)

    def body(*refs):
        i = pl.program_id(0)
        xr = refs[0:nt]
        wr = refs[nt:2 * nt]
        pos = 2 * nt
        affr = []
        for t in range(nt):
            if terms[t][3] is not None:
                affr.append((refs[pos], refs[pos + 1]))
                pos += 2
            else:
                affr.append(None)
        y_ref, st_ref, acc_ref = refs[pos], refs[pos + 1], refs[pos + 2]
        y = jnp.zeros((BM, C), jnp.float32)
        for t in range(nt):
            xv = xr[t][...]
            if affr[t] is not None:
                s, b = affr[t]
                xv = jnp.maximum(xv * s[...] + b[...], 0.0)
            y = y + jnp.dot(xv, wr[t][...],
                            preferred_element_type=jnp.float32)
        y_ref[...] = y
        rows = i * BM + lax.broadcasted_iota(jnp.int32, (BM, 1), 0)
        ym = jnp.where(rows < n_valid, y, 0.0)

        @pl.when(i == 0)
        def _():
            acc_ref[...] = jnp.zeros_like(acc_ref)

        acc_ref[0, :] += jnp.sum(ym, axis=0)
        acc_ref[1, :] += jnp.sum(ym * ym, axis=0)

        @pl.when(i == NB - 1)
        def _():
            st_ref[...] = acc_ref[...]

    in_specs = []
    args = []
    for (x, off, w, aff) in terms:
        in_specs.append(pl.BlockSpec((BM, C), lambda i, o=off: (i + o, 0)))
        args.append(x)
    for (x, off, w, aff) in terms:
        in_specs.append(pl.BlockSpec((C, C), lambda i: (0, 0)))
        args.append(w)
    for (x, off, w, aff) in terms:
        if aff is not None:
            in_specs.append(pl.BlockSpec((1, C), lambda i: (0, 0)))
            in_specs.append(pl.BlockSpec((1, C), lambda i: (0, 0)))
            args.append(aff[0])
            args.append(aff[1])

    return pl.pallas_call(
        body,
        grid=(NB,),
        in_specs=in_specs,
        out_specs=[pl.BlockSpec((BM, C), lambda i: (i, 0)),
                   pl.BlockSpec((2, C), lambda i: (0, 0))],
        out_shape=[jax.ShapeDtypeStruct((NP, C), jnp.float32),
                   jax.ShapeDtypeStruct((2, C), jnp.float32)],
        scratch_shapes=[pltpu.VMEM((2, C), jnp.float32)],
    )(*args)


def _affine_relu_ext(x, s, b):
    bm = NP2 // 16

    def body(x_ref, s_ref, b_ref, o_ref):
        i = pl.program_id(0)
        rows = i * bm + lax.broadcasted_iota(jnp.int32, (bm, 1), 0)
        y = jnp.maximum(x_ref[...] * s_ref[...] + b_ref[...], 0.0)
        o_ref[...] = jnp.where(rows < NP, y, 0.0)

    return pl.pallas_call(
        body,
        grid=(16,),
        in_specs=[pl.BlockSpec((bm, C), lambda i: (i, 0)),
                  pl.BlockSpec((1, C), lambda i: (0, 0)),
                  pl.BlockSpec((1, C), lambda i: (0, 0))],
        out_specs=pl.BlockSpec((bm, C), lambda i: (i, 0)),
        out_shape=jax.ShapeDtypeStruct((NP2, C), jnp.float32),
    )(x, s, b)


def _affine_relu_head(x, s, b):
    bm = N // 5

    def body(x_ref, s_ref, b_ref, o_ref):
        o_ref[...] = jnp.maximum(x_ref[...] * s_ref[...] + b_ref[...], 0.0)

    return pl.pallas_call(
        body,
        grid=(5,),
        in_specs=[pl.BlockSpec((bm, C), lambda i: (i, 0)),
                  pl.BlockSpec((1, C), lambda i: (0, 0)),
                  pl.BlockSpec((1, C), lambda i: (0, 0))],
        out_specs=pl.BlockSpec((bm, C), lambda i: (i, 0)),
        out_shape=jax.ShapeDtypeStruct((N, C), jnp.float32),
    )(x, s, b)


def _ew_affine_relu(x, s, b, bm):
    rows = x.shape[0]
    nb = rows // bm

    def body(x_ref, s_ref, b_ref, o_ref):
        o_ref[...] = jnp.maximum(x_ref[...] * s_ref[...] + b_ref[...], 0.0)

    return pl.pallas_call(
        body,
        grid=(nb,),
        in_specs=[pl.BlockSpec((bm, C), lambda i: (i, 0)),
                  pl.BlockSpec((1, C), lambda i: (0, 0)),
                  pl.BlockSpec((1, C), lambda i: (0, 0))],
        out_specs=pl.BlockSpec((bm, C), lambda i: (i, 0)),
        out_shape=jax.ShapeDtypeStruct((rows, C), jnp.float32),
    )(x, s, b)


FLW = 72
FL = 64 * FLW
XL = 4768


def _conv_stats(x2d, xden, wr):
    def body(x2_ref, xd_ref, w_ref, y_ref, st_ref, acc_ref, xs_ref):
        i = pl.program_id(0)
        xs_ref[...] = jnp.zeros((XL, 256), jnp.float32)

        def cp(y_row, _):
            dst = (y_row + 1) * FLW + 8
            xs_ref[pl.ds(dst, W), 0:C] = x2_ref[0, pl.ds(y_row * W, W), :]
            xs_ref[pl.ds(dst, W), C:2 * C] = xd_ref[0, pl.ds(y_row * W, W), :]
            return 0

        lax.fori_loop(0, H, cp, 0)
        y = jnp.zeros((FL, C), jnp.float32)
        for dy in range(3):
            for dx in range(3):
                off = FLW * dy + dx + 7
                t = dy * 3 + dx
                xs = xs_ref[pl.ds(off, FL), :]
                y = y + jnp.dot(xs, w_ref[pl.ds(t * 256, 256), :],
                                preferred_element_type=jnp.float32)
        y_ref[0] = y
        col = lax.broadcasted_iota(jnp.int32, (FL, C), 0) % FLW
        ym = jnp.where(col < 64, y, 0.0)

        @pl.when(i == 0)
        def _():
            acc_ref[...] = jnp.zeros_like(acc_ref)

        acc_ref[0, :] += jnp.sum(ym, axis=0)
        acc_ref[1, :] += jnp.sum(ym * ym, axis=0)

        @pl.when(i == 5)
        def _():
            st_ref[...] = acc_ref[...]

    return pl.pallas_call(
        body,
        grid=(6,),
        in_specs=[pl.BlockSpec((1, H * W, C), lambda i: (i, 0, 0)),
                  pl.BlockSpec((1, H * W, C), lambda i: (i, 0, 0)),
                  pl.BlockSpec((2304, C), lambda i: (0, 0))],
        out_specs=[pl.BlockSpec((1, FL, C), lambda i: (i, 0, 0)),
                   pl.BlockSpec((2, C), lambda i: (0, 0))],
        out_shape=[jax.ShapeDtypeStruct((6, FL, C), jnp.float32),
                   jax.ShapeDtypeStruct((2, C), jnp.float32)],
        scratch_shapes=[pltpu.VMEM((2, C), jnp.float32),
                        pltpu.VMEM((XL, 256), jnp.float32)],
    )(x2d, xden, wr)


def _finalize(st, cnt, g, b):
    mean = st[0] / cnt
    var = st[1] / cnt - mean * mean
    scale = g / jnp.sqrt(var + 1e-5)
    bias = b - mean * scale
    return scale.reshape(1, C), bias.reshape(1, C)


def kernel(feat_2d_all, feat_3d_F, links, W_sep, g_sep, b_sep,
           W_vf1, g_vf1, b_vf1, W_vf2, g_vf2, b_vf2,
           W_f3d, g_f3d, b_f3d, W_conv, g_2d, b_2d):
    f32 = jnp.float32
    f2d_nhwc = feat_2d_all.reshape(VIEWS * B, C, H, W).transpose(0, 2, 3, 1)
    f2v_ext = jnp.concatenate(
        [f2d_nhwc.reshape(SLOTS, C), jnp.zeros((128, C), f32)], axis=0)
    f3d_pad = jnp.concatenate(
        [feat_3d_F, jnp.zeros((NP - N, C), f32)], axis=0)

    varr = jnp.arange(VIEWS, dtype=jnp.int32)[None, :]
    bi = links[:, 0, :]
    hi = links[:, 1, :]
    wi = links[:, 2, :]
    fl = links[:, 3, :]
    slot = ((varr * B + bi) * H + hi) * W + wi
    zrow = GPAD + (jnp.arange(N, dtype=jnp.int32)[:, None] % 128)
    gidx = jnp.where(fl == 1, slot, zrow)

    slot_t = jnp.full((VIEWS, NP), PAD_SLOT, jnp.int32)
    slot_t = slot_t.at[:, :N].set(slot.T).reshape(-1)
    zfill = GPAD + (jnp.arange(GTOT, dtype=jnp.int32) % 128)
    gidx_flat = zfill.at[:VIEWS * NP].set(
        jnp.where(
            jnp.zeros((VIEWS, NP), jnp.bool_).at[:, :N].set(True),
            jnp.zeros((VIEWS, NP), jnp.int32).at[:, :N].set(gidx.T),
            zfill[:VIEWS * NP].reshape(VIEWS, NP)).reshape(-1))
    gidx_rs = jnp.pad(gidx_flat.reshape(NWORK, 37, 128),
                      ((0, 0), (0, 11), (0, 0)), constant_values=GPAD)

    y_sep, st_sep = _mm_stats([(f3d_pad, 0, W_sep, None)], N)
    s_sep, o_sep = _finalize(st_sep, N, g_sep, b_sep)
    a_ext = _affine_relu_ext(y_sep, s_sep, o_sep)

    init = jnp.full((SLOTS,), -1, jnp.int32)
    w32 = _winner_phase1(slot_t, init).reshape(NWORK, SLOTS)
    dense2d = _winner_reduce_gather(w32, a_ext)
    gact = _sc_gather(f2v_ext, gidx_rs)

    wv = W_vf1.reshape(VIEWS, C, C)
    y_vf1, st1 = _mm_stats(
        [(gact, 0, wv[0], None), (gact, NB, wv[1], None),
         (gact, 2 * NB, wv[2], None)], N)
    s1, o1 = _finalize(st1, N, g_vf1, b_vf1)
    y_vf2, st2 = _mm_stats([(y_vf1, 0, W_vf2, (s1, o1))], N)
    s2, o2 = _finalize(st2, N, g_vf2, b_vf2)
    wf = W_f3d.reshape(2, C, C)
    y_f3d, st3 = _mm_stats(
        [(f3d_pad, 0, wf[0], None), (y_vf2, 0, wf[1], (s2, o2))], N)
    s3, o3 = _finalize(st3, N, g_f3d, b_f3d)
    fused_3d = _affine_relu_head(y_f3d, s3, o3)

    wr = W_conv.transpose(2, 3, 1, 0).reshape(9 * 2 * C, C)
    y_cw, stc = _conv_stats(f2d_nhwc.reshape(VIEWS * B, H * W, C),
                            dense2d.reshape(VIEWS * B, H * W, C), wr)
    sc_, oc_ = _finalize(stc, 6 * H * W, g_2d, b_2d)
    y2 = _ew_affine_relu(y_cw.reshape(6 * FL, C), sc_, oc_, FL)
    fused_2d = (y2.reshape(VIEWS * B, H, FLW, C)[:, :, :64, :]
                .transpose(0, 3, 1, 2))
    return fused_3d, fused_2d
